# baseline (speedup 1.0000x reference)
import jax
import jax.numpy as jnp
from jax.experimental import pallas as pl
from jax.experimental.pallas import tpu as pltpu

_K = 128
_HN = 512
_BMM = 1024
_BM = 64
_BISECT_ITERS = 20


def _matmul_body(x_ref, w_ref, b_ref, h_ref):
    i = pl.program_id(1)
    bmm = h_ref.shape[0]
    xb = x_ref[pl.ds(i * bmm, bmm), :]
    z = jax.lax.dot_general(
        xb, w_ref[...], (((1,), (1,)), ((), ())),
        preferred_element_type=jnp.float32,
        precision=jax.lax.Precision.DEFAULT,
    )
    z = z + b_ref[...]
    h_ref[...] = 1.0 / (1.0 + jnp.exp(-z))


def _select_body(h_ref, o_ref):
    h = h_ref[...]
    bm, dh = h.shape
    kf = jnp.float32(_K)

    def count_iter(arr):
        def it(_, lohi):
            lo, hi = lohi
            mid = 0.5 * (lo + hi)
            cnt = jnp.sum((arr >= mid).astype(jnp.float32), axis=1,
                          keepdims=True)
            big = cnt >= kf
            return jnp.where(big, mid, lo), jnp.where(big, hi, mid)
        return it

    cm = jnp.max(h.reshape(bm, dh // 128, 128), axis=2)
    lo0 = jnp.zeros((bm, 1), jnp.float32)
    hi0 = jnp.ones((bm, 1), jnp.float32)
    t1, _ = jax.lax.fori_loop(0, 30, count_iter(cm), (lo0, hi0))
    gm = jnp.max(cm, axis=1, keepdims=True) + jnp.float32(2.0 ** -18)

    lo, _ = jax.lax.fori_loop(0, _BISECT_ITERS, count_iter(h), (t1, gm))
    o_ref[...] = jnp.where(h >= lo, h, 0.0)


def kernel(x, W, b):
    B, DIN = x.shape
    DH = W.shape[0]
    hn = min(_HN, DH)
    bmm = min(_BMM, B)
    bm = min(_BM, B)
    b2 = b.reshape(1, DH)

    hidden = pl.pallas_call(
        _matmul_body,
        grid=(DH // hn, B // bmm),
        in_specs=[
            pl.BlockSpec((B, DIN), lambda j, i: (0, 0)),
            pl.BlockSpec((hn, DIN), lambda j, i: (j, 0)),
            pl.BlockSpec((1, hn), lambda j, i: (0, j)),
        ],
        out_specs=pl.BlockSpec((bmm, hn), lambda j, i: (i, j)),
        out_shape=jax.ShapeDtypeStruct((B, DH), jnp.float32),
        compiler_params=pltpu.CompilerParams(
            dimension_semantics=("arbitrary", "arbitrary"),
            vmem_limit_bytes=60 * 1024 * 1024,
        ),
    )(x, W, b2)

    out = pl.pallas_call(
        _select_body,
        grid=(B // bm,),
        in_specs=[pl.BlockSpec((bm, DH), lambda i: (i, 0))],
        out_specs=pl.BlockSpec((bm, DH), lambda i: (i, 0)),
        out_shape=jax.ShapeDtypeStruct((B, DH), jnp.float32),
        input_output_aliases={0: 0},
        compiler_params=pltpu.CompilerParams(
            dimension_semantics=("parallel",),
            vmem_limit_bytes=60 * 1024 * 1024,
        ),
    )(hidden)
    return out


# speedup vs baseline: 2.5979x; 2.5979x over previous
import jax
import jax.numpy as jnp
from jax.experimental import pallas as pl
from jax.experimental.pallas import tpu as pltpu

_K = 128
_HN = 512
_BMM = 1024
_BM = 64
_BISECT_ITERS = 20


def _matmul_body(x_ref, w_ref, b_ref, h_ref):
    i = pl.program_id(1)
    bmm = h_ref.shape[0]
    xb = x_ref[pl.ds(i * bmm, bmm), :]
    z = jax.lax.dot_general(
        xb, w_ref[...], (((1,), (1,)), ((), ())),
        preferred_element_type=jnp.float32,
        precision=jax.lax.Precision.DEFAULT,
    )
    z = z + b_ref[...]
    h_ref[...] = 1.0 / (1.0 + jnp.exp(-z))


def _select_body(h_ref, o_ref):
    h = h_ref[...]
    bm, dh = h.shape
    kf = jnp.float32(_K)

    def it(_, lohi):
        lo, hi = lohi
        mid = 0.5 * (lo + hi)
        cnt = jnp.sum((h >= mid).astype(jnp.float32), axis=1, keepdims=True)
        big = cnt >= kf
        return jnp.where(big, mid, lo), jnp.where(big, hi, mid)

    m = h
    while m.shape[1] > _K:
        s = m.shape[1] // 2
        m = jnp.maximum(m[:, :s], m[:, s:])
    t1 = jnp.min(m, axis=1, keepdims=True)
    gm = jnp.max(m, axis=1, keepdims=True) + jnp.float32(2.0 ** -18)

    lo, _ = jax.lax.fori_loop(0, _BISECT_ITERS, it, (t1, gm))
    o_ref[...] = jnp.where(h >= lo, h, 0.0)


def kernel(x, W, b):
    B, DIN = x.shape
    DH = W.shape[0]
    hn = min(_HN, DH)
    bmm = min(_BMM, B)
    bm = min(_BM, B)
    b2 = b.reshape(1, DH)

    hidden = pl.pallas_call(
        _matmul_body,
        grid=(DH // hn, B // bmm),
        in_specs=[
            pl.BlockSpec((B, DIN), lambda j, i: (0, 0)),
            pl.BlockSpec((hn, DIN), lambda j, i: (j, 0)),
            pl.BlockSpec((1, hn), lambda j, i: (0, j)),
        ],
        out_specs=pl.BlockSpec((bmm, hn), lambda j, i: (i, j)),
        out_shape=jax.ShapeDtypeStruct((B, DH), jnp.float32),
        compiler_params=pltpu.CompilerParams(
            dimension_semantics=("arbitrary", "arbitrary"),
            vmem_limit_bytes=60 * 1024 * 1024,
        ),
    )(x, W, b2)

    out = pl.pallas_call(
        _select_body,
        grid=(B // bm,),
        in_specs=[pl.BlockSpec((bm, DH), lambda i: (i, 0))],
        out_specs=pl.BlockSpec((bm, DH), lambda i: (i, 0)),
        out_shape=jax.ShapeDtypeStruct((B, DH), jnp.float32),
        input_output_aliases={0: 0},
        compiler_params=pltpu.CompilerParams(
            dimension_semantics=("parallel",),
            vmem_limit_bytes=60 * 1024 * 1024,
        ),
    )(hidden)
    return out
